# baseline (speedup 1.0000x reference)
import jax
import jax.numpy as jnp
from jax.experimental import pallas as pl
from jax.experimental.pallas import tpu as pltpu

_SIZES = [240, 480, 960, 1920] + [2000] * 46 + [800] + [1920, 960, 480, 240]
_OFFS = [sum(_SIZES[:i]) for i in range(len(_SIZES))]
_NCHUNK = len(_SIZES)
_MAX_CHUNK = max(_SIZES)
_NBUF = 8
_LA = 4


def _ring_copy(ent_in, rel_in, ent_out, rel_out, buf, rbuf,
               lsem, ssem, rlsem, rssem):
    rel_load = pltpu.make_async_copy(rel_in, rbuf, rlsem)
    rel_store = pltpu.make_async_copy(rbuf, rel_out, rssem)
    rel_load.start()

    def load(j):
        sl = pl.ds(_OFFS[j], _SIZES[j])
        return pltpu.make_async_copy(
            ent_in.at[sl], buf.at[j % _NBUF, pl.ds(0, _SIZES[j])],
            lsem.at[j % _NBUF])

    def store(j):
        sl = pl.ds(_OFFS[j], _SIZES[j])
        return pltpu.make_async_copy(
            buf.at[j % _NBUF, pl.ds(0, _SIZES[j])], ent_out.at[sl],
            ssem.at[j % _NBUF])

    for j in range(_LA):
        load(j).start()

    rel_load.wait()
    rel_store.start()

    for i in range(_NCHUNK):
        load(i).wait()
        store(i).start()
        j = i + _LA
        if j < _NCHUNK:
            if j >= _NBUF:
                store(j - _NBUF).wait()
            load(j).start()

    for k in range(_NCHUNK - _NBUF, _NCHUNK):
        store(k).wait()
    rel_store.wait()


def kernel(emb_ent, emb_rel, edge_index, rel, edge_index_all, rel_all):
    return tuple(pl.pallas_call(
        _ring_copy,
        in_specs=[pl.BlockSpec(memory_space=pl.ANY),
                  pl.BlockSpec(memory_space=pl.ANY)],
        out_specs=[pl.BlockSpec(memory_space=pl.ANY),
                   pl.BlockSpec(memory_space=pl.ANY)],
        out_shape=[jax.ShapeDtypeStruct(emb_ent.shape, emb_ent.dtype),
                   jax.ShapeDtypeStruct(emb_rel.shape, emb_rel.dtype)],
        scratch_shapes=[
            pltpu.VMEM((_NBUF, _MAX_CHUNK, 768), jnp.float32),
            pltpu.VMEM(emb_rel.shape, jnp.float32),
            pltpu.SemaphoreType.DMA((_NBUF,)),
            pltpu.SemaphoreType.DMA((_NBUF,)),
            pltpu.SemaphoreType.DMA,
            pltpu.SemaphoreType.DMA,
        ],
    )(emb_ent, emb_rel))


# speedup vs baseline: 1.0031x; 1.0031x over previous
import jax
import jax.numpy as jnp
from jax.experimental import pallas as pl
from jax.experimental.pallas import tpu as pltpu

_ENT_BLOCK = 4000


def _copy_both(ent_ref, rel_ref, ent_out, rel_out):
    ent_out[...] = ent_ref[...]

    @pl.when(pl.program_id(0) == 0)
    def _():
        rel_out[...] = rel_ref[...]


def kernel(emb_ent, emb_rel, edge_index, rel, edge_index_all, rel_all):
    n, d = emb_ent.shape
    m, r = emb_rel.shape
    ent_out, rel_out = pl.pallas_call(
        _copy_both,
        grid=(n // _ENT_BLOCK,),
        in_specs=[pl.BlockSpec((_ENT_BLOCK, d), lambda i: (i, 0)),
                  pl.BlockSpec((m, r), lambda i: (0, 0))],
        out_specs=[pl.BlockSpec((_ENT_BLOCK, d), lambda i: (i, 0)),
                   pl.BlockSpec((m, r), lambda i: (0, 0))],
        out_shape=[jax.ShapeDtypeStruct((n, d), emb_ent.dtype),
                   jax.ShapeDtypeStruct((m, r), emb_rel.dtype)],
        compiler_params=pltpu.CompilerParams(
            dimension_semantics=("arbitrary",)),
    )(emb_ent, emb_rel)
    return (ent_out, rel_out)
